# baseline (speedup 1.0000x reference)
import jax
import jax.numpy as jnp
from jax.experimental import pallas as pl
from jax.experimental.pallas import tpu as pltpu

ALPHA = 0.2
N = 512
F = 64
B = 2
NBLK = 4
BLK = N // NBLK


def _leaky_relu(v):
    return jnp.maximum(v, ALPHA * v)


def _gatcell_kernel(x_ref, adj_ref, w1_ref, a1_ref, w2_ref, a2_ref, out_ref,
                    h1_s, f2t_s, h2_s, z_s, g2t_s, adj_s):
    i = pl.program_id(0)

    @pl.when(i == 0)
    def _init():
        w1eff = w1_ref[:F, :] + w1_ref[F:, :]
        for b in range(B):
            h1 = jnp.dot(x_ref[b], w1eff,
                         preferred_element_type=jnp.float32)
            h1_s[b] = h1
            f2 = jnp.dot(h1, a1_ref[2 * F:, :],
                         preferred_element_type=jnp.float32)
            f2t_s[b:b + 1, :] = f2.reshape(1, N)

    @pl.when(i < NBLK)
    def _layer1():
        start = i * BLK
        adj_blk = adj_ref[...]
        adj_s[pl.ds(start, BLK), :] = adj_blk
        for b in range(B):
            h1 = h1_s[b]
            h1r = h1_s[b, pl.ds(start, BLK), :]
            f1 = jnp.dot(h1r, a1_ref[:2 * F, :],
                         preferred_element_type=jnp.float32)
            p = adj_blk * jnp.exp(_leaky_relu(f1 + f2t_s[b:b + 1, :]))
            s = jnp.sum(p, axis=1, keepdims=True)
            gv = jnp.dot(p, h1, preferred_element_type=jnp.float32) / s
            r = jax.nn.sigmoid(gv[:, :F])
            z = jax.nn.sigmoid(gv[:, F:])
            z_s[b, pl.ds(start, BLK), :] = z
            xr = x_ref[b, pl.ds(start, BLK), :]
            h2_s[b, pl.ds(start, BLK), :] = (
                jnp.dot(xr, w2_ref[:F, :], preferred_element_type=jnp.float32)
                + jnp.dot(r * xr, w2_ref[F:, :],
                          preferred_element_type=jnp.float32))

    @pl.when(i == NBLK)
    def _mid():
        for b in range(B):
            g2 = jnp.dot(h2_s[b], a2_ref[F:, :],
                         preferred_element_type=jnp.float32)
            g2t_s[b:b + 1, :] = g2.reshape(1, N)

    @pl.when(i >= NBLK)
    def _layer2():
        start = (i - NBLK) * BLK
        adj_blk = adj_s[pl.ds(start, BLK), :]
        for b in range(B):
            h2 = h2_s[b]
            h2r = h2_s[b, pl.ds(start, BLK), :]
            g1 = jnp.dot(h2r, a2_ref[:F, :],
                         preferred_element_type=jnp.float32)
            p = adj_blk * jnp.exp(_leaky_relu(g1 + g2t_s[b:b + 1, :]))
            s = jnp.sum(p, axis=1, keepdims=True)
            hp = jnp.dot(p, h2, preferred_element_type=jnp.float32) / s
            t = jnp.tanh(hp)
            z = z_s[b, pl.ds(start, BLK), :]
            xr = x_ref[b, pl.ds(start, BLK), :]
            out_ref[b] = t + z * (xr - t)


def kernel(X, adj, W1, a1, W2, a2):
    return pl.pallas_call(
        _gatcell_kernel,
        grid=(2 * NBLK,),
        in_specs=[
            pl.BlockSpec((B, N, F), lambda i: (0, 0, 0)),
            pl.BlockSpec((BLK, N), lambda i: (jnp.minimum(i, NBLK - 1), 0)),
            pl.BlockSpec(W1.shape, lambda i: (0, 0)),
            pl.BlockSpec(a1.shape, lambda i: (0, 0)),
            pl.BlockSpec(W2.shape, lambda i: (0, 0)),
            pl.BlockSpec(a2.shape, lambda i: (0, 0)),
        ],
        out_specs=pl.BlockSpec((B, BLK, F),
                               lambda i: (0, jnp.maximum(i - NBLK, 0), 0)),
        scratch_shapes=[
            pltpu.VMEM((B, N, 2 * F), jnp.float32),
            pltpu.VMEM((B, N), jnp.float32),
            pltpu.VMEM((B, N, F), jnp.float32),
            pltpu.VMEM((B, N, F), jnp.float32),
            pltpu.VMEM((B, N), jnp.float32),
            pltpu.VMEM((N, N), jnp.float32),
        ],
        out_shape=jax.ShapeDtypeStruct(X.shape, X.dtype),
        compiler_params=pltpu.CompilerParams(
            dimension_semantics=("arbitrary",),
        ),
    )(X, adj, W1, a1, W2, a2)


# speedup vs baseline: 1.3153x; 1.3153x over previous
import jax
import jax.numpy as jnp
from jax.experimental import pallas as pl
from jax.experimental.pallas import tpu as pltpu

ALPHA = 0.2
N = 512
F = 64
B = 2
NBLK = 4
BLK = N // NBLK


def _leaky_relu(v):
    return jnp.maximum(v, ALPHA * v)


def _attention(hs, adj_blocks, a_lo, a_hi):
    f1s = [jnp.dot(h, a_lo, preferred_element_type=jnp.float32) for h in hs]
    f2ts = [jnp.dot(h, a_hi,
                    preferred_element_type=jnp.float32).reshape(1, N)
            for h in hs]
    outs = []
    for b, (h, f1, f2t) in enumerate(zip(hs, f1s, f2ts)):
        blocks = []
        for k in range(NBLK):
            f1k = f1[k * BLK:(k + 1) * BLK]
            p = adj_blocks[k] * jnp.exp(_leaky_relu(f1k + f2t))
            s = jnp.sum(p, axis=1, keepdims=True)
            num = jnp.dot(p, h, preferred_element_type=jnp.float32)
            blocks.append(num / s)
        outs.append(jnp.concatenate(blocks, axis=0))
    return outs


def _gatcell_kernel(x_ref, aq0, aq1, aq2, aq3,
                    w1_ref, a1_ref, w2_ref, a2_ref, out_ref):
    adj_blocks = [aq0[...], aq1[...], aq2[...], aq3[...]]
    xs = [x_ref[b] for b in range(B)]

    w1eff = w1_ref[:F, :] + w1_ref[F:, :]
    h1s = [jnp.dot(x, w1eff, preferred_element_type=jnp.float32) for x in xs]
    gvs = _attention(h1s, adj_blocks, a1_ref[:2 * F, :], a1_ref[2 * F:, :])

    rs_zs = [(jax.nn.sigmoid(gv[:, :F]), jax.nn.sigmoid(gv[:, F:]))
             for gv in gvs]
    h2s = [jnp.dot(x, w2_ref[:F, :], preferred_element_type=jnp.float32)
           + jnp.dot(r * x, w2_ref[F:, :], preferred_element_type=jnp.float32)
           for x, (r, _) in zip(xs, rs_zs)]
    hps = _attention(h2s, adj_blocks, a2_ref[:F, :], a2_ref[F:, :])

    for b, (x, (_, z), hp) in enumerate(zip(xs, rs_zs, hps)):
        t = jnp.tanh(hp)
        out_ref[b] = t + z * (x - t)


def kernel(X, adj, W1, a1, W2, a2):
    adj_specs = [pl.BlockSpec((BLK, N), lambda i, k=k: (k, 0))
                 for k in range(NBLK)]
    return pl.pallas_call(
        _gatcell_kernel,
        grid=(1,),
        in_specs=[pl.BlockSpec(X.shape, lambda i: (0, 0, 0))] + adj_specs + [
            pl.BlockSpec(W1.shape, lambda i: (0, 0)),
            pl.BlockSpec(a1.shape, lambda i: (0, 0)),
            pl.BlockSpec(W2.shape, lambda i: (0, 0)),
            pl.BlockSpec(a2.shape, lambda i: (0, 0)),
        ],
        out_specs=pl.BlockSpec(X.shape, lambda i: (0, 0, 0)),
        out_shape=jax.ShapeDtypeStruct(X.shape, X.dtype),
    )(X, adj, adj, adj, adj, W1, a1, W2, a2)


# speedup vs baseline: 1.3966x; 1.0618x over previous
import jax
import jax.numpy as jnp
from jax.experimental import pallas as pl

ALPHA = 0.2
N = 512
F = 64
B = 2
LOG2E = 1.4426950408889634


def _attention(hs, adj, a_lo, a_hi):
    outs = []
    for h in hs:
        f1 = jnp.dot(h, a_lo, preferred_element_type=jnp.float32)
        f2t = jnp.dot(h, a_hi,
                      preferred_element_type=jnp.float32).reshape(1, N)
        v = f1 + f2t
        p = adj * jnp.exp2(jnp.maximum(v, ALPHA * v))
        he = jnp.concatenate(
            [h, jnp.ones((N, 1), jnp.float32)], axis=1)
        num = jnp.dot(p, he, preferred_element_type=jnp.float32)
        outs.append(num[:, :-1] / num[:, -1:])
    return outs


def _gatcell_kernel(x_ref, adj_ref, w1_ref, a1_ref, w2_ref, a2_ref, out_ref):
    adj = adj_ref[...]
    xs = [x_ref[b] for b in range(B)]

    w1eff = w1_ref[:F, :] + w1_ref[F:, :]
    h1s = [jnp.dot(x, w1eff, preferred_element_type=jnp.float32) for x in xs]
    gvs = _attention(h1s, adj, LOG2E * a1_ref[:2 * F, :],
                     LOG2E * a1_ref[2 * F:, :])

    rs_zs = [(jax.nn.sigmoid(gv[:, :F]), jax.nn.sigmoid(gv[:, F:]))
             for gv in gvs]
    h2s = [jnp.dot(x, w2_ref[:F, :], preferred_element_type=jnp.float32)
           + jnp.dot(r * x, w2_ref[F:, :], preferred_element_type=jnp.float32)
           for x, (r, _) in zip(xs, rs_zs)]
    hps = _attention(h2s, adj, LOG2E * a2_ref[:F, :], LOG2E * a2_ref[F:, :])

    for b, (x, (_, z), hp) in enumerate(zip(xs, rs_zs, hps)):
        t = jnp.tanh(hp)
        out_ref[b] = t + z * (x - t)


def kernel(X, adj, W1, a1, W2, a2):
    return pl.pallas_call(
        _gatcell_kernel,
        out_shape=jax.ShapeDtypeStruct(X.shape, X.dtype),
    )(X, adj, W1, a1, W2, a2)


# speedup vs baseline: 1.4120x; 1.0110x over previous
import jax
import jax.numpy as jnp
from jax.experimental import pallas as pl

ALPHA = 0.2
N = 512
F = 64
B = 2
LOG2E = 1.4426950408889634


def _attention(hs, adj, a_lo, a_hi):
    a_both = jnp.concatenate([a_lo, a_hi], axis=1)
    ones_col = jnp.ones((N, 1), jnp.float32)
    outs = []
    for h in hs:
        ff = jnp.dot(h, a_both, preferred_element_type=jnp.float32)
        lhs = jnp.concatenate([ff[:, 0:1], ones_col], axis=1)
        rhs = jnp.concatenate([ones_col, ff[:, 1:2]], axis=1)
        v = jax.lax.dot_general(lhs, rhs, (((1,), (1,)), ((), ())),
                                preferred_element_type=jnp.float32)
        p = adj * jnp.exp2(jnp.maximum(v, ALPHA * v))
        he = jnp.concatenate([h, ones_col], axis=1)
        num = jnp.dot(p, he, preferred_element_type=jnp.float32)
        outs.append(num[:, :-1] * (1.0 / num[:, -1:]))
    return outs


def _gatcell_kernel(x_ref, adj_ref, w1_ref, a1_ref, w2_ref, a2_ref, out_ref):
    adj = adj_ref[...]
    xs = [x_ref[b] for b in range(B)]

    w1eff = w1_ref[:F, :] + w1_ref[F:, :]
    h1s = [jnp.dot(x, w1eff, preferred_element_type=jnp.float32) for x in xs]
    gvs = _attention(h1s, adj, LOG2E * a1_ref[:2 * F, :],
                     LOG2E * a1_ref[2 * F:, :])

    rs_zs = [(jax.nn.sigmoid(gv[:, :F]), jax.nn.sigmoid(gv[:, F:]))
             for gv in gvs]
    h2s = [jnp.dot(x, w2_ref[:F, :], preferred_element_type=jnp.float32)
           + jnp.dot(r * x, w2_ref[F:, :], preferred_element_type=jnp.float32)
           for x, (r, _) in zip(xs, rs_zs)]
    hps = _attention(h2s, adj, LOG2E * a2_ref[:F, :], LOG2E * a2_ref[F:, :])

    for b, (x, (_, z), hp) in enumerate(zip(xs, rs_zs, hps)):
        t = jnp.tanh(hp)
        out_ref[b] = t + z * (x - t)


def kernel(X, adj, W1, a1, W2, a2):
    return pl.pallas_call(
        _gatcell_kernel,
        out_shape=jax.ShapeDtypeStruct(X.shape, X.dtype),
    )(X, adj, W1, a1, W2, a2)
